# Initial kernel scaffold; baseline (speedup 1.0000x reference)
import jax
import jax.numpy as jnp
from jax.experimental import pallas as pl


def kernel(x):
    raise NotImplementedError("write your pallas kernel here")


# speedup vs baseline: 3.3079x; 3.3079x over previous
import jax
import jax.numpy as jnp
from jax import lax
from jax.experimental import pallas as pl
from jax.experimental.pallas import tpu as pltpu

_K = 64
_N = 32768
_BLOCK_R = 8


def _wta_block(x_ref, o_ref):
    x = x_ref[...]
    ix = lax.bitcast_convert_type(x, jnp.int32)
    key = jnp.where(ix < 0, ix ^ jnp.int32(0x7FFFFFFF), ix)

    def count_ge(m):
        return jnp.sum((key >= m).astype(jnp.int32), axis=1, keepdims=True)

    n_pos = count_ge(jnp.zeros((x.shape[0], 1), jnp.int32))
    pos = n_pos >= _K
    lo = jnp.where(pos, jnp.int32(0), jnp.int32(-(2**31)))
    hi = jnp.where(pos, jnp.int32(2**31 - 1), jnp.int32(-1))

    def body(_, carry):
        lo, hi = carry
        mid = lo + lax.shift_right_logical(hi - lo, 1) + 1
        ge = count_ge(mid) >= _K
        return jnp.where(ge, mid, lo), jnp.where(ge, hi, mid - 1)

    lo, hi = lax.fori_loop(0, 31, body, (lo, hi), unroll=False)
    kstar = lo

    gt = key > kstar
    eq = key == kstar
    n_gt = jnp.sum(gt.astype(jnp.int32), axis=1, keepdims=True)
    needed = _K - n_gt

    idx = lax.broadcasted_iota(jnp.int32, x.shape, 1)
    ilo = jnp.zeros_like(n_gt)
    ihi = jnp.full_like(n_gt, _N - 1)

    def ibody(_, carry):
        ilo, ihi = carry
        mid = ilo + lax.shift_right_logical(ihi - ilo, 1)
        cnt = jnp.sum((eq & (idx <= mid)).astype(jnp.int32), axis=1,
                      keepdims=True)
        ok = cnt >= needed
        return jnp.where(ok, ilo, mid + 1), jnp.where(ok, mid, ihi)

    ilo, ihi = lax.fori_loop(0, 15, ibody, (ilo, ihi), unroll=False)
    cstar = ilo

    sel = gt | (eq & (idx <= cstar))
    o_ref[...] = jnp.where(sel, x, 0.0)


def kernel(x):
    B, N = x.shape
    grid = (B // _BLOCK_R,)
    return pl.pallas_call(
        _wta_block,
        grid=grid,
        in_specs=[pl.BlockSpec((_BLOCK_R, N), lambda i: (i, 0))],
        out_specs=pl.BlockSpec((_BLOCK_R, N), lambda i: (i, 0)),
        out_shape=jax.ShapeDtypeStruct((B, N), x.dtype),
    )(x)


# speedup vs baseline: 10.4463x; 3.1579x over previous
import jax
import jax.numpy as jnp
from jax import lax
from jax.experimental import pallas as pl
from jax.experimental.pallas import tpu as pltpu

_K = 64
_N = 32768
_BLOCK_R = 8
_TOPM = 8
_NEG_INF_KEY = -(2**31)


def _count_ge(key, m):
    return jnp.sum((key >= m).astype(jnp.int32), axis=1, keepdims=True)


def _search_kth(key, k):
    n_pos = _count_ge(key, jnp.zeros((key.shape[0], 1), jnp.int32))
    pos = n_pos >= k
    lo = jnp.where(pos, jnp.int32(0), jnp.int32(-(2**31)))
    hi = jnp.where(pos, jnp.int32(2**31 - 1), jnp.int32(-1))

    def body(_, carry):
        lo, hi = carry
        mid = lo + lax.shift_right_logical(hi - lo, 1) + 1
        ge = _count_ge(key, mid) >= k
        return jnp.where(ge, mid, lo), jnp.where(ge, hi, mid - 1)

    lo, hi = lax.fori_loop(0, 31, body, (lo, hi), unroll=False)
    return lo


def _wta_block(x_ref, o_ref):
    x = x_ref[...]
    R = x.shape[0]
    ix = lax.bitcast_convert_type(x, jnp.int32)
    key = jnp.where(ix < 0, ix ^ jnp.int32(0x7FFFFFFF), ix)

    key3 = key.reshape(R, _N // 128, 128)
    work = key3
    cands = []
    for _ in range(_TOPM):
        cm = jnp.max(work, axis=1, keepdims=True)
        cands.append(cm)
        work = jnp.where(work == cm, jnp.int32(_NEG_INF_KEY), work)
    cand = jnp.concatenate(cands, axis=1).reshape(R, _TOPM * 128)

    t_cand = _search_kth(cand, _K)

    n_gt0 = _count_ge(key, t_cand + 1)
    n_ge0 = _count_ge(key, t_cand)
    ok = jnp.all((n_ge0 >= _K) & (n_gt0 < _K))

    def exact_path():
        t = _search_kth(key, _K)
        return t, _count_ge(key, t + 1), _count_ge(key, t)

    kstar, n_gt, n_ge = lax.cond(
        ok, lambda: (t_cand, n_gt0, n_ge0), exact_path)

    no_ties = jnp.all(n_ge == _K)

    def out_simple():
        return jnp.where(key >= kstar, x, 0.0)

    def out_ties():
        eq = key == kstar
        needed = _K - n_gt
        idx = lax.broadcasted_iota(jnp.int32, key.shape, 1)
        ilo = jnp.zeros((R, 1), jnp.int32)
        ihi = jnp.full((R, 1), _N - 1, jnp.int32)

        def ibody(_, carry):
            ilo, ihi = carry
            mid = ilo + lax.shift_right_logical(ihi - ilo, 1)
            cnt = jnp.sum((eq & (idx <= mid)).astype(jnp.int32), axis=1,
                          keepdims=True)
            take = cnt >= needed
            return jnp.where(take, ilo, mid + 1), jnp.where(take, mid, ihi)

        ilo, ihi = lax.fori_loop(0, 15, ibody, (ilo, ihi), unroll=False)
        sel = (key > kstar) | (eq & (idx <= ilo))
        return jnp.where(sel, x, 0.0)

    o_ref[...] = lax.cond(no_ties, out_simple, out_ties)


def kernel(x):
    B, N = x.shape
    grid = (B // _BLOCK_R,)
    return pl.pallas_call(
        _wta_block,
        grid=grid,
        in_specs=[pl.BlockSpec((_BLOCK_R, N), lambda i: (i, 0))],
        out_specs=pl.BlockSpec((_BLOCK_R, N), lambda i: (i, 0)),
        out_shape=jax.ShapeDtypeStruct((B, N), x.dtype),
    )(x)


# speedup vs baseline: 17.2567x; 1.6519x over previous
import jax
import jax.numpy as jnp
from jax import lax
from jax.experimental import pallas as pl
from jax.experimental.pallas import tpu as pltpu

_K = 64
_N = 32768
_BLOCK_R = 32
_TOPM = 8


def _count_ge(key, m):
    return jnp.sum((key >= m).astype(jnp.int32), axis=1, keepdims=True)


def _search_kth(key, k):
    n_pos = _count_ge(key, jnp.zeros((key.shape[0], 1), jnp.int32))
    pos = n_pos >= k
    lo = jnp.where(pos, jnp.int32(0), jnp.int32(-(2**31)))
    hi = jnp.where(pos, jnp.int32(2**31 - 1), jnp.int32(-1))

    def body(_, carry):
        lo, hi = carry
        mid = lo + lax.shift_right_logical(hi - lo, 1) + 1
        ge = _count_ge(key, mid) >= k
        return jnp.where(ge, mid, lo), jnp.where(ge, hi, mid - 1)

    lo, hi = lax.fori_loop(0, 31, body, (lo, hi), unroll=False)
    return lo


def _wta_block(x_ref, o_ref):
    x = x_ref[...]
    R = x.shape[0]
    ix = lax.bitcast_convert_type(x, jnp.int32)
    key = jnp.where(ix < 0, ix ^ jnp.int32(0x7FFFFFFF), ix)

    key3 = key.reshape(R, _N // 128, 128)
    work = key3
    cands = []
    for _ in range(_TOPM):
        cm = jnp.max(work, axis=1, keepdims=True)
        cands.append(cm)
        work = jnp.where(work == cm, jnp.int32(-(2**31)), work)
    cand = jnp.concatenate(cands, axis=1).reshape(R, _TOPM * 128)

    t_cand = _search_kth(cand, _K)

    ge = key >= t_cand
    n_ge = jnp.sum(ge.astype(jnp.int32), axis=1, keepdims=True)
    n_gt = _count_ge(key, t_cand + 1)
    exact_no_ties = jnp.all((n_ge == _K) & (n_gt < _K))

    def fast_path():
        return jnp.where(ge, x, 0.0)

    def slow_path():
        ok = jnp.all((n_ge >= _K) & (n_gt < _K))
        kstar = lax.cond(ok, lambda: t_cand, lambda: _search_kth(key, _K))
        gt = key > kstar
        eq = key == kstar
        needed = _K - jnp.sum(gt.astype(jnp.int32), axis=1, keepdims=True)
        idx = lax.broadcasted_iota(jnp.int32, key.shape, 1)
        ilo = jnp.zeros((R, 1), jnp.int32)
        ihi = jnp.full((R, 1), _N - 1, jnp.int32)

        def ibody(_, carry):
            ilo, ihi = carry
            mid = ilo + lax.shift_right_logical(ihi - ilo, 1)
            cnt = jnp.sum((eq & (idx <= mid)).astype(jnp.int32), axis=1,
                          keepdims=True)
            take = cnt >= needed
            return jnp.where(take, ilo, mid + 1), jnp.where(take, mid, ihi)

        ilo, _ = lax.fori_loop(0, 15, ibody, (ilo, ihi), unroll=False)
        sel = gt | (eq & (idx <= ilo))
        return jnp.where(sel, x, 0.0)

    o_ref[...] = lax.cond(exact_no_ties, fast_path, slow_path)


def kernel(x):
    B, N = x.shape
    grid = (B // _BLOCK_R,)
    return pl.pallas_call(
        _wta_block,
        grid=grid,
        in_specs=[pl.BlockSpec((_BLOCK_R, N), lambda i: (i, 0))],
        out_specs=pl.BlockSpec((_BLOCK_R, N), lambda i: (i, 0)),
        out_shape=jax.ShapeDtypeStruct((B, N), x.dtype),
    )(x)


# speedup vs baseline: 27.5914x; 1.5989x over previous
import jax
import jax.numpy as jnp
from jax import lax
from jax.experimental import pallas as pl
from jax.experimental.pallas import tpu as pltpu

_K = 64
_N = 32768
_BLOCK_R = 32
_CELL_M = 4
_LANE_M = 8


def _to_key(v):
    iv = lax.bitcast_convert_type(v, jnp.int32)
    return jnp.where(iv < 0, iv ^ jnp.int32(0x7FFFFFFF), iv)


def _count_ge(key, m):
    return jnp.sum((key >= m).astype(jnp.int32), axis=1, keepdims=True)


def _search_kth(key, k):
    n_pos = _count_ge(key, jnp.zeros((key.shape[0], 1), jnp.int32))
    pos = n_pos >= k
    lo = jnp.where(pos, jnp.int32(0), jnp.int32(-(2**31)))
    hi = jnp.where(pos, jnp.int32(2**31 - 1), jnp.int32(-1))

    def body(_, carry):
        lo, hi = carry
        mid = lo + lax.shift_right_logical(hi - lo, 1) + 1
        ge = _count_ge(key, mid) >= k
        return jnp.where(ge, mid, lo), jnp.where(ge, hi, mid - 1)

    lo, hi = lax.fori_loop(0, 31, body, (lo, hi), unroll=False)
    return lo


def _wta_block(x_ref, o_ref):
    x = x_ref[...]
    R = x.shape[0]
    neg = jnp.float32(-jnp.inf)

    work = x.reshape(R, 32, 8, 128)
    c0 = []
    for _ in range(_CELL_M):
        cm = jnp.max(work, axis=1, keepdims=True)
        c0.append(cm)
        work = jnp.where(work == cm, neg, work)
    cand0 = jnp.concatenate(c0, axis=1).reshape(R, _CELL_M * 8, 128)

    c1 = []
    for _ in range(_LANE_M):
        cm = jnp.max(cand0, axis=1, keepdims=True)
        c1.append(cm)
        cand0 = jnp.where(cand0 == cm, neg, cand0)
    cand = jnp.concatenate(c1, axis=1).reshape(R, _LANE_M * 128)

    t_key = _search_kth(_to_key(cand), _K)
    t_f = lax.bitcast_convert_type(
        jnp.where(t_key < 0, t_key ^ jnp.int32(0x7FFFFFFF), t_key),
        jnp.float32)

    ge = x >= t_f
    n_ge = jnp.sum(ge.astype(jnp.int32), axis=1, keepdims=True)
    fast = jnp.all(n_ge == _K)

    @pl.when(fast)
    def _fast_path():
        o_ref[...] = jnp.where(ge, x, 0.0)

    @pl.when(jnp.logical_not(fast))
    def _slow_path():
        key = _to_key(x)
        kstar = _search_kth(key, _K)
        gt = key > kstar
        eq = key == kstar
        needed = _K - jnp.sum(gt.astype(jnp.int32), axis=1, keepdims=True)
        idx = lax.broadcasted_iota(jnp.int32, key.shape, 1)
        ilo = jnp.zeros((R, 1), jnp.int32)
        ihi = jnp.full((R, 1), _N - 1, jnp.int32)

        def ibody(_, carry):
            ilo, ihi = carry
            mid = ilo + lax.shift_right_logical(ihi - ilo, 1)
            cnt = jnp.sum((eq & (idx <= mid)).astype(jnp.int32), axis=1,
                          keepdims=True)
            take = cnt >= needed
            return jnp.where(take, ilo, mid + 1), jnp.where(take, mid, ihi)

        ilo, _ = lax.fori_loop(0, 15, ibody, (ilo, ihi), unroll=False)
        sel = gt | (eq & (idx <= ilo))
        o_ref[...] = jnp.where(sel, x, 0.0)


def kernel(x):
    B, N = x.shape
    grid = (B // _BLOCK_R,)
    return pl.pallas_call(
        _wta_block,
        grid=grid,
        in_specs=[pl.BlockSpec((_BLOCK_R, N), lambda i: (i, 0))],
        out_specs=pl.BlockSpec((_BLOCK_R, N), lambda i: (i, 0)),
        out_shape=jax.ShapeDtypeStruct((B, N), x.dtype),
    )(x)


# speedup vs baseline: 29.2791x; 1.0612x over previous
import jax
import jax.numpy as jnp
from jax import lax
from jax.experimental import pallas as pl
from jax.experimental.pallas import tpu as pltpu

_K = 64
_N = 32768
_BLOCK_R = 32
_CELL_M = 4
_LANE_M = 8


def _to_key(v):
    iv = lax.bitcast_convert_type(v, jnp.int32)
    return jnp.where(iv < 0, iv ^ jnp.int32(0x7FFFFFFF), iv)


def _count_ge(key, m):
    return jnp.sum((key >= m).astype(jnp.int32), axis=1, keepdims=True)


def _search_kth(key, k):
    n_pos = _count_ge(key, jnp.zeros((key.shape[0], 1), jnp.int32))
    pos = n_pos >= k
    lo = jnp.where(pos, jnp.int32(0), jnp.int32(-(2**31)))
    hi = jnp.where(pos, jnp.int32(2**31 - 1), jnp.int32(-1))

    def body(_, carry):
        lo, hi = carry
        mid = lo + lax.shift_right_logical(hi - lo, 1) + 1
        ge = _count_ge(key, mid) >= k
        return jnp.where(ge, mid, lo), jnp.where(ge, hi, mid - 1)

    lo, hi = lax.fori_loop(0, 31, body, (lo, hi), unroll=False)
    return lo


def _wta_block(x_ref, o_ref):
    x = x_ref[...]
    R = x.shape[0]
    neg = jnp.float32(-jnp.inf)

    xv = x.reshape(R, 32, 8, 128)
    ta = [jnp.full((R, 1, 8, 128), neg) for _ in range(_CELL_M)]
    for s in range(32):
        v = xv[:, s:s + 1]
        for j in range(_CELL_M):
            nt = jnp.maximum(ta[j], v)
            if j < _CELL_M - 1:
                v = jnp.minimum(ta[j], v)
            ta[j] = nt
    cand0 = jnp.concatenate(ta, axis=1).reshape(R, _CELL_M * 8, 128)

    c1 = []
    for _ in range(_LANE_M):
        cm = jnp.max(cand0, axis=1, keepdims=True)
        c1.append(cm)
        cand0 = jnp.where(cand0 == cm, neg, cand0)
    cand = jnp.concatenate(c1, axis=1).reshape(R, _LANE_M * 128)

    t_key = _search_kth(_to_key(cand), _K)
    t_f = lax.bitcast_convert_type(
        jnp.where(t_key < 0, t_key ^ jnp.int32(0x7FFFFFFF), t_key),
        jnp.float32)

    ge = x >= t_f
    n_ge = jnp.sum(ge.astype(jnp.int32), axis=1, keepdims=True)
    fast = jnp.all(n_ge == _K)

    @pl.when(fast)
    def _fast_path():
        o_ref[...] = jnp.where(ge, x, 0.0)

    @pl.when(jnp.logical_not(fast))
    def _slow_path():
        key = _to_key(x)
        kstar = _search_kth(key, _K)
        gt = key > kstar
        eq = key == kstar
        needed = _K - jnp.sum(gt.astype(jnp.int32), axis=1, keepdims=True)
        idx = lax.broadcasted_iota(jnp.int32, key.shape, 1)
        ilo = jnp.zeros((R, 1), jnp.int32)
        ihi = jnp.full((R, 1), _N - 1, jnp.int32)

        def ibody(_, carry):
            ilo, ihi = carry
            mid = ilo + lax.shift_right_logical(ihi - ilo, 1)
            cnt = jnp.sum((eq & (idx <= mid)).astype(jnp.int32), axis=1,
                          keepdims=True)
            take = cnt >= needed
            return jnp.where(take, ilo, mid + 1), jnp.where(take, mid, ihi)

        ilo, _ = lax.fori_loop(0, 15, ibody, (ilo, ihi), unroll=False)
        sel = gt | (eq & (idx <= ilo))
        o_ref[...] = jnp.where(sel, x, 0.0)


def kernel(x):
    B, N = x.shape
    grid = (B // _BLOCK_R,)
    return pl.pallas_call(
        _wta_block,
        grid=grid,
        in_specs=[pl.BlockSpec((_BLOCK_R, N), lambda i: (i, 0))],
        out_specs=pl.BlockSpec((_BLOCK_R, N), lambda i: (i, 0)),
        out_shape=jax.ShapeDtypeStruct((B, N), x.dtype),
    )(x)
